# baseline (speedup 1.0000x reference)
import functools

import jax
import jax.numpy as jnp
from jax import lax
from jax.experimental import pallas as pl
from jax.experimental.pallas import tpu as pltpu
from jax.experimental.pallas import tpu_sc as plsc

EPS = 1e-5
HIDDEN = 64
HID_P = 128

_NC = 2
_NS = 16
_NW = _NC * _NS
_CHUNK = 80
_NBUF = 5


def _sc_gather(idx, table_p, n_rows):
    b_per_w = n_rows // _NW
    n_chunk = b_per_w // _CHUNK
    n_batch = n_chunk // _NBUF
    mesh = plsc.VectorSubcoreMesh(core_axis_name="c", subcore_axis_name="s")

    @functools.partial(
        pl.kernel,
        mesh=mesh,
        out_type=jax.ShapeDtypeStruct((n_rows, HID_P), jnp.float32),
        scratch_types=[
            pltpu.VMEM((b_per_w,), jnp.int32),
            pltpu.VMEM((_NBUF, _CHUNK, HID_P), jnp.float32),
            pltpu.SemaphoreType.DMA,
            pltpu.SemaphoreType.DMA,
        ],
        compiler_params=pltpu.CompilerParams(use_tc_tiling_on_sc=True),
    )
    def gather_kernel(idx_hbm, table_hbm, out_hbm, idx_v, bufs, sem_g, sem_w):
        wid = lax.axis_index("s") * _NC + lax.axis_index("c")
        base = wid * b_per_w
        pltpu.sync_copy(idx_hbm.at[pl.ds(base, b_per_w)], idx_v)

        def batch(i, _):
            goff = pl.multiple_of(i * (_NBUF * _CHUNK), 8)

            @pl.when(i > 0)
            def _():
                for b in range(_NBUF):
                    pltpu.make_async_copy(
                        out_hbm.at[pl.ds(base, _CHUNK)], bufs.at[b], sem_w
                    ).wait()

            gs = []
            for b in range(_NBUF):
                off = pl.multiple_of(goff + b * _CHUNK, 8)
                gs.append(
                    pltpu.async_copy(
                        table_hbm.at[idx_v.at[pl.ds(off, _CHUNK)]],
                        bufs.at[b],
                        sem_g,
                    )
                )
            for g in gs:
                g.wait()
            for b in range(_NBUF):
                off = pl.multiple_of(goff + b * _CHUNK, 8)
                pltpu.async_copy(
                    bufs.at[b], out_hbm.at[pl.ds(base + off, _CHUNK)], sem_w
                )
            return 0

        lax.fori_loop(0, n_batch, batch, 0)
        for b in range(_NBUF):
            pltpu.make_async_copy(
                out_hbm.at[pl.ds(base, _CHUNK)], bufs.at[b], sem_w
            ).wait()

    return gather_kernel(idx, table_p)


def _lnmm_body(h_ref, lnw_ref, lnb_ref, w_ref, b_ref, o_ref):
    lblk = o_ref.shape[0]
    bsz = o_ref.shape[2]
    for s in range(lblk):
        hb = h_ref[pl.ds(s * bsz, bsz), :]
        mean = jnp.sum(hb, axis=1, keepdims=True) * (1.0 / HIDDEN)
        c = hb - mean
        lane = lax.broadcasted_iota(jnp.int32, hb.shape, 1)
        c = jnp.where(lane < HIDDEN, c, 0.0)
        var = jnp.sum(c * c, axis=1, keepdims=True) * (1.0 / HIDDEN)
        hn = c * lax.rsqrt(var + EPS)
        hn = hn * lnw_ref[...] + lnb_ref[...]
        acc = lax.dot_general(
            w_ref[...], hn, (((1,), (1,)), ((), ())),
            preferred_element_type=jnp.float32,
        )
        o_ref[s] = acc + b_ref[...]


def _lnmm_body2(prev_ref, h_ref, lnw_ref, lnb_ref, w_ref, b_ref, o_ref):
    del prev_ref
    _lnmm_body(h_ref, lnw_ref, lnb_ref, w_ref, b_ref, o_ref)


def _tc_lnmm(h_t, lnw_p, lnb_p, fc_w_p, fc_bc, seq, bsz, lblk, l0, out_prev):
    vocab = fc_w_p.shape[0]
    n_l = h_t.shape[0] // bsz
    grid = (n_l // lblk,)
    common_in = [
        pl.BlockSpec((lblk * bsz, HID_P), lambda l: (l, 0)),
        pl.BlockSpec((1, HID_P), lambda l: (0, 0)),
        pl.BlockSpec((1, HID_P), lambda l: (0, 0)),
        pl.BlockSpec((vocab, HID_P), lambda l: (0, 0)),
        pl.BlockSpec((vocab, 1), lambda l: (0, 0)),
    ]
    lb0 = l0 // lblk
    out_spec = pl.BlockSpec((lblk, vocab, bsz), lambda l: (l + lb0, 0, 0))
    out_shape = jax.ShapeDtypeStruct((seq, vocab, bsz), jnp.float32)
    if out_prev is None:
        return pl.pallas_call(
            _lnmm_body,
            grid=grid,
            in_specs=common_in,
            out_specs=out_spec,
            out_shape=out_shape,
        )(h_t, lnw_p, lnb_p, fc_w_p, fc_bc)
    return pl.pallas_call(
        _lnmm_body2,
        grid=grid,
        in_specs=[pl.BlockSpec(memory_space=pl.ANY)] + common_in,
        out_specs=out_spec,
        out_shape=out_shape,
        input_output_aliases={0: 0},
    )(out_prev, h_t, lnw_p, lnb_p, fc_w_p, fc_bc)


def kernel(x, emb_table, ln_w, ln_b, fc_w, fc_b):
    bsz, seq = x.shape
    pad = HID_P - HIDDEN
    table_p = jnp.pad(emb_table, ((0, 0), (0, pad)))
    fc_w_p = jnp.pad(fc_w, ((0, 0), (0, pad)))
    lnw_p = jnp.pad(ln_w.reshape(1, -1), ((0, 0), (0, pad)))
    lnb_p = jnp.pad(ln_b.reshape(1, -1), ((0, 0), (0, pad)))
    fc_bc = fc_b.reshape(-1, 1)
    idx_t = x.T.reshape(-1)
    half = seq // 2
    h_a = _sc_gather(idx_t[: half * bsz], table_p, half * bsz)
    h_b = _sc_gather(idx_t[half * bsz :], table_p, (seq - half) * bsz)
    out_a = _tc_lnmm(
        h_a, lnw_p, lnb_p, fc_w_p, fc_bc, seq, bsz, lblk=5, l0=0, out_prev=None
    )
    out_t = _tc_lnmm(
        h_b, lnw_p, lnb_p, fc_w_p, fc_bc, seq, bsz, lblk=5, l0=half,
        out_prev=out_a,
    )
    return out_t.transpose(2, 0, 1)


# speedup vs baseline: 1.0050x; 1.0050x over previous
import functools

import jax
import jax.numpy as jnp
from jax import lax
from jax.experimental import pallas as pl
from jax.experimental.pallas import tpu as pltpu
from jax.experimental.pallas import tpu_sc as plsc

EPS = 1e-5
HIDDEN = 64
HID_P = 128

_NC = 2
_NS = 16
_NW = _NC * _NS
_CHUNK = 80
_NBUF = 5


def _sc_gather(idx, table_p, n_rows):
    b_per_w = n_rows // _NW
    n_chunk = b_per_w // _CHUNK
    n_batch = n_chunk // _NBUF
    mesh = plsc.VectorSubcoreMesh(core_axis_name="c", subcore_axis_name="s")

    @functools.partial(
        pl.kernel,
        mesh=mesh,
        out_type=jax.ShapeDtypeStruct((n_rows, HID_P), jnp.float32),
        scratch_types=[
            pltpu.VMEM((b_per_w,), jnp.int32),
            pltpu.VMEM((_NBUF, _CHUNK, HID_P), jnp.float32),
            pltpu.SemaphoreType.DMA,
            pltpu.SemaphoreType.DMA,
        ],
        compiler_params=pltpu.CompilerParams(use_tc_tiling_on_sc=True),
    )
    def gather_kernel(idx_hbm, table_hbm, out_hbm, idx_v, bufs, sem_g, sem_w):
        wid = lax.axis_index("s") * _NC + lax.axis_index("c")
        base = wid * b_per_w
        pltpu.sync_copy(idx_hbm.at[pl.ds(base, b_per_w)], idx_v)

        def batch(i, _):
            goff = pl.multiple_of(i * (_NBUF * _CHUNK), 8)

            @pl.when(i > 0)
            def _():
                for b in range(_NBUF):
                    pltpu.make_async_copy(
                        out_hbm.at[pl.ds(base, _CHUNK)], bufs.at[b], sem_w
                    ).wait()

            gs = []
            for b in range(_NBUF):
                off = pl.multiple_of(goff + b * _CHUNK, 8)
                gs.append(
                    pltpu.async_copy(
                        table_hbm.at[idx_v.at[pl.ds(off, _CHUNK)]],
                        bufs.at[b],
                        sem_g,
                    )
                )
            for g in gs:
                g.wait()
            for b in range(_NBUF):
                off = pl.multiple_of(goff + b * _CHUNK, 8)
                pltpu.async_copy(
                    bufs.at[b], out_hbm.at[pl.ds(base + off, _CHUNK)], sem_w
                )
            return 0

        lax.fori_loop(0, n_batch, batch, 0)
        for b in range(_NBUF):
            pltpu.make_async_copy(
                out_hbm.at[pl.ds(base, _CHUNK)], bufs.at[b], sem_w
            ).wait()

    return gather_kernel(idx, table_p)


def _lnmm_body(h_ref, lnw_ref, lnb_ref, w_ref, b_ref, o_ref):
    lblk = o_ref.shape[0]
    bsz = o_ref.shape[2]
    for s in range(lblk):
        hb = h_ref[pl.ds(s * bsz, bsz), :]
        mean = jnp.sum(hb, axis=1, keepdims=True) * (1.0 / HIDDEN)
        c = hb - mean
        lane = lax.broadcasted_iota(jnp.int32, hb.shape, 1)
        c = jnp.where(lane < HIDDEN, c, 0.0)
        var = jnp.sum(c * c, axis=1, keepdims=True) * (1.0 / HIDDEN)
        hn = c * lax.rsqrt(var + EPS)
        hn = hn * lnw_ref[...] + lnb_ref[...]
        acc = lax.dot_general(
            w_ref[...], hn, (((1,), (1,)), ((), ())),
            preferred_element_type=jnp.float32,
        )
        o_ref[s] = acc + b_ref[...]


def _lnmm_body2(prev_ref, h_ref, lnw_ref, lnb_ref, w_ref, b_ref, o_ref):
    del prev_ref
    _lnmm_body(h_ref, lnw_ref, lnb_ref, w_ref, b_ref, o_ref)


def _tc_lnmm(h_t, lnw_p, lnb_p, fc_w_p, fc_bc, seq, bsz, lblk, l0, out_prev):
    vocab = fc_w_p.shape[0]
    n_l = h_t.shape[0] // bsz
    grid = (n_l // lblk,)
    common_in = [
        pl.BlockSpec((lblk * bsz, HID_P), lambda l: (l, 0)),
        pl.BlockSpec((1, HID_P), lambda l: (0, 0)),
        pl.BlockSpec((1, HID_P), lambda l: (0, 0)),
        pl.BlockSpec((vocab, HID_P), lambda l: (0, 0)),
        pl.BlockSpec((vocab, 1), lambda l: (0, 0)),
    ]
    lb0 = l0 // lblk
    out_spec = pl.BlockSpec((lblk, vocab, bsz), lambda l: (l + lb0, 0, 0))
    out_shape = jax.ShapeDtypeStruct((seq, vocab, bsz), jnp.float32)
    if out_prev is None:
        return pl.pallas_call(
            _lnmm_body,
            grid=grid,
            in_specs=common_in,
            out_specs=out_spec,
            out_shape=out_shape,
        )(h_t, lnw_p, lnb_p, fc_w_p, fc_bc)
    return pl.pallas_call(
        _lnmm_body2,
        grid=grid,
        in_specs=[pl.BlockSpec(memory_space=pl.ANY)] + common_in,
        out_specs=out_spec,
        out_shape=out_shape,
        input_output_aliases={0: 0},
    )(out_prev, h_t, lnw_p, lnb_p, fc_w_p, fc_bc)


def kernel(x, emb_table, ln_w, ln_b, fc_w, fc_b):
    bsz, seq = x.shape
    pad = HID_P - HIDDEN
    table_p = jnp.pad(emb_table, ((0, 0), (0, pad)))
    fc_w_p = jnp.pad(fc_w, ((0, 0), (0, pad)))
    lnw_p = jnp.pad(ln_w.reshape(1, -1), ((0, 0), (0, pad)))
    lnb_p = jnp.pad(ln_b.reshape(1, -1), ((0, 0), (0, pad)))
    fc_bc = fc_b.reshape(-1, 1)
    idx_t = x.T.reshape(-1)
    half = seq // 2
    h_a = _sc_gather(idx_t[: half * bsz], table_p, half * bsz)
    out_a = _tc_lnmm(
        h_a, lnw_p, lnb_p, fc_w_p, fc_bc, seq, bsz, lblk=5, l0=0, out_prev=None
    )
    h_b = _sc_gather(idx_t[half * bsz :], table_p, (seq - half) * bsz)
    out_t = _tc_lnmm(
        h_b, lnw_p, lnb_p, fc_w_p, fc_bc, seq, bsz, lblk=5, l0=half,
        out_prev=out_a,
    )
    return out_t.transpose(2, 0, 1)


# speedup vs baseline: 1.0137x; 1.0086x over previous
import functools

import jax
import jax.numpy as jnp
from jax import lax
from jax.experimental import pallas as pl
from jax.experimental.pallas import tpu as pltpu
from jax.experimental.pallas import tpu_sc as plsc

EPS = 1e-5
HIDDEN = 64
HID_P = 128

_NC = 2
_NS = 16
_NW = _NC * _NS
_CHUNK = 80
_NBUF = 5


def _sc_gather(idx, table_p, n_rows):
    dtype = table_p.dtype
    b_per_w = n_rows // _NW
    n_chunk = b_per_w // _CHUNK
    n_batch = n_chunk // _NBUF
    mesh = plsc.VectorSubcoreMesh(core_axis_name="c", subcore_axis_name="s")

    @functools.partial(
        pl.kernel,
        mesh=mesh,
        out_type=jax.ShapeDtypeStruct((n_rows, HID_P), dtype),
        scratch_types=[
            pltpu.VMEM((b_per_w,), jnp.int32),
            pltpu.VMEM((_NBUF, _CHUNK, HID_P), dtype),
            pltpu.SemaphoreType.DMA,
            pltpu.SemaphoreType.DMA,
        ],
        compiler_params=pltpu.CompilerParams(use_tc_tiling_on_sc=True),
    )
    def gather_kernel(idx_hbm, table_hbm, out_hbm, idx_v, bufs, sem_g, sem_w):
        wid = lax.axis_index("s") * _NC + lax.axis_index("c")
        base = wid * b_per_w
        pltpu.sync_copy(idx_hbm.at[pl.ds(base, b_per_w)], idx_v)

        def batch(i, _):
            goff = pl.multiple_of(i * (_NBUF * _CHUNK), 8)

            @pl.when(i > 0)
            def _():
                for b in range(_NBUF):
                    pltpu.make_async_copy(
                        out_hbm.at[pl.ds(base, _CHUNK)], bufs.at[b], sem_w
                    ).wait()

            gs = []
            for b in range(_NBUF):
                off = pl.multiple_of(goff + b * _CHUNK, 8)
                gs.append(
                    pltpu.async_copy(
                        table_hbm.at[idx_v.at[pl.ds(off, _CHUNK)]],
                        bufs.at[b],
                        sem_g,
                    )
                )
            for g in gs:
                g.wait()
            for b in range(_NBUF):
                off = pl.multiple_of(goff + b * _CHUNK, 8)
                pltpu.async_copy(
                    bufs.at[b], out_hbm.at[pl.ds(base + off, _CHUNK)], sem_w
                )
            return 0

        lax.fori_loop(0, n_batch, batch, 0)
        for b in range(_NBUF):
            pltpu.make_async_copy(
                out_hbm.at[pl.ds(base, _CHUNK)], bufs.at[b], sem_w
            ).wait()

    return gather_kernel(idx, table_p)


def _lnmm_body(h_ref, lnw_ref, lnb_ref, w_ref, b_ref, o_ref):
    lblk = o_ref.shape[0]
    bsz = o_ref.shape[2]
    for s in range(lblk):
        hb = h_ref[pl.ds(s * bsz, bsz), :].astype(jnp.float32)
        mean = jnp.sum(hb, axis=1, keepdims=True) * (1.0 / HIDDEN)
        c = hb - mean
        lane = lax.broadcasted_iota(jnp.int32, hb.shape, 1)
        c = jnp.where(lane < HIDDEN, c, 0.0)
        var = jnp.sum(c * c, axis=1, keepdims=True) * (1.0 / HIDDEN)
        hn = c * lax.rsqrt(var + EPS)
        hn = hn * lnw_ref[...] + lnb_ref[...]
        acc = lax.dot_general(
            w_ref[...], hn, (((1,), (1,)), ((), ())),
            preferred_element_type=jnp.float32,
        )
        o_ref[s] = acc + b_ref[...]


def _lnmm_body2(prev_ref, h_ref, lnw_ref, lnb_ref, w_ref, b_ref, o_ref):
    del prev_ref
    _lnmm_body(h_ref, lnw_ref, lnb_ref, w_ref, b_ref, o_ref)


def _tc_lnmm(h_t, lnw_p, lnb_p, fc_w_p, fc_bc, seq, bsz, lblk, l0, out_prev):
    vocab = fc_w_p.shape[0]
    n_l = h_t.shape[0] // bsz
    grid = (n_l // lblk,)
    common_in = [
        pl.BlockSpec((lblk * bsz, HID_P), lambda l: (l, 0)),
        pl.BlockSpec((1, HID_P), lambda l: (0, 0)),
        pl.BlockSpec((1, HID_P), lambda l: (0, 0)),
        pl.BlockSpec((vocab, HID_P), lambda l: (0, 0)),
        pl.BlockSpec((vocab, 1), lambda l: (0, 0)),
    ]
    lb0 = l0 // lblk
    out_spec = pl.BlockSpec((lblk, vocab, bsz), lambda l: (l + lb0, 0, 0))
    out_shape = jax.ShapeDtypeStruct((seq, vocab, bsz), jnp.float32)
    if out_prev is None:
        return pl.pallas_call(
            _lnmm_body,
            grid=grid,
            in_specs=common_in,
            out_specs=out_spec,
            out_shape=out_shape,
        )(h_t, lnw_p, lnb_p, fc_w_p, fc_bc)
    return pl.pallas_call(
        _lnmm_body2,
        grid=grid,
        in_specs=[pl.BlockSpec(memory_space=pl.ANY)] + common_in,
        out_specs=out_spec,
        out_shape=out_shape,
        input_output_aliases={0: 0},
    )(out_prev, h_t, lnw_p, lnb_p, fc_w_p, fc_bc)


def kernel(x, emb_table, ln_w, ln_b, fc_w, fc_b):
    bsz, seq = x.shape
    pad = HID_P - HIDDEN
    table_p = jnp.pad(emb_table, ((0, 0), (0, pad)))
    fc_w_p = jnp.pad(fc_w, ((0, 0), (0, pad)))
    lnw_p = jnp.pad(ln_w.reshape(1, -1), ((0, 0), (0, pad)))
    lnb_p = jnp.pad(ln_b.reshape(1, -1), ((0, 0), (0, pad)))
    fc_bc = fc_b.reshape(-1, 1)
    idx_t = x.T.reshape(-1)
    h_t = _sc_gather(idx_t, table_p, seq * bsz)
    out_t = _tc_lnmm(
        h_t, lnw_p, lnb_p, fc_w_p, fc_bc, seq, bsz, lblk=5, l0=0, out_prev=None
    )
    return out_t.transpose(2, 0, 1)


# speedup vs baseline: 1.0143x; 1.0006x over previous
import functools

import jax
import jax.numpy as jnp
from jax import lax
from jax.experimental import pallas as pl
from jax.experimental.pallas import tpu as pltpu
from jax.experimental.pallas import tpu_sc as plsc

EPS = 1e-5
HIDDEN = 64
HID_P = 128

_NC = 2
_NS = 16
_NW = _NC * _NS
_CHUNK = 80
_NBUF = 5


def _sc_gather(idx, table_p, n_rows):
    dtype = table_p.dtype
    b_per_w = n_rows // _NW
    n_chunk = b_per_w // _CHUNK
    n_batch = n_chunk // _NBUF
    mesh = plsc.VectorSubcoreMesh(core_axis_name="c", subcore_axis_name="s")

    @functools.partial(
        pl.kernel,
        mesh=mesh,
        out_type=jax.ShapeDtypeStruct((n_rows, HID_P), dtype),
        scratch_types=[
            pltpu.VMEM((b_per_w,), jnp.int32),
            pltpu.VMEM((2, _NBUF, _CHUNK, HID_P), dtype),
            pltpu.SemaphoreType.DMA,
            pltpu.SemaphoreType.DMA,
        ],
        compiler_params=pltpu.CompilerParams(use_tc_tiling_on_sc=True),
    )
    def gather_kernel(idx_hbm, table_hbm, out_hbm, idx_v, bufs, sem_g, sem_w):
        wid = lax.axis_index("s") * _NC + lax.axis_index("c")
        base = wid * b_per_w
        pltpu.sync_copy(idx_hbm.at[pl.ds(base, b_per_w)], idx_v)

        def batch(i, _):
            grp = lax.rem(i, 2)
            goff = pl.multiple_of(i * (_NBUF * _CHUNK), 8)

            @pl.when(i >= 2)
            def _():
                for b in range(_NBUF):
                    pltpu.make_async_copy(
                        out_hbm.at[pl.ds(base, _CHUNK)], bufs.at[0, b], sem_w
                    ).wait()

            gs = []
            for b in range(_NBUF):
                off = pl.multiple_of(goff + b * _CHUNK, 8)
                gs.append(
                    pltpu.async_copy(
                        table_hbm.at[idx_v.at[pl.ds(off, _CHUNK)]],
                        bufs.at[grp, b],
                        sem_g,
                    )
                )
            for g in gs:
                g.wait()
            for b in range(_NBUF):
                off = pl.multiple_of(goff + b * _CHUNK, 8)
                pltpu.async_copy(
                    bufs.at[grp, b], out_hbm.at[pl.ds(base + off, _CHUNK)], sem_w
                )
            return 0

        lax.fori_loop(0, n_batch, batch, 0)
        for b in range(2 * _NBUF if n_batch >= 2 else _NBUF):
            pltpu.make_async_copy(
                out_hbm.at[pl.ds(base, _CHUNK)], bufs.at[0, b % _NBUF], sem_w
            ).wait()

    return gather_kernel(idx, table_p)


def _lnmm_body(h_ref, lnw_ref, lnb_ref, w_ref, b_ref, o_ref):
    lblk = o_ref.shape[0]
    bsz = o_ref.shape[2]
    for s in range(lblk):
        hb = h_ref[pl.ds(s * bsz, bsz), :].astype(jnp.float32)
        mean = jnp.sum(hb, axis=1, keepdims=True) * (1.0 / HIDDEN)
        c = hb - mean
        lane = lax.broadcasted_iota(jnp.int32, hb.shape, 1)
        c = jnp.where(lane < HIDDEN, c, 0.0)
        var = jnp.sum(c * c, axis=1, keepdims=True) * (1.0 / HIDDEN)
        hn = c * lax.rsqrt(var + EPS)
        hn = hn * lnw_ref[...] + lnb_ref[...]
        acc = lax.dot_general(
            w_ref[...], hn, (((1,), (1,)), ((), ())),
            preferred_element_type=jnp.float32,
        )
        o_ref[s] = acc + b_ref[...]


def _lnmm_body2(prev_ref, h_ref, lnw_ref, lnb_ref, w_ref, b_ref, o_ref):
    del prev_ref
    _lnmm_body(h_ref, lnw_ref, lnb_ref, w_ref, b_ref, o_ref)


def _tc_lnmm(h_t, lnw_p, lnb_p, fc_w_p, fc_bc, seq, bsz, lblk, l0, out_prev):
    vocab = fc_w_p.shape[0]
    n_l = h_t.shape[0] // bsz
    grid = (n_l // lblk,)
    common_in = [
        pl.BlockSpec((lblk * bsz, HID_P), lambda l: (l, 0)),
        pl.BlockSpec((1, HID_P), lambda l: (0, 0)),
        pl.BlockSpec((1, HID_P), lambda l: (0, 0)),
        pl.BlockSpec((vocab, HID_P), lambda l: (0, 0)),
        pl.BlockSpec((vocab, 1), lambda l: (0, 0)),
    ]
    lb0 = l0 // lblk
    out_spec = pl.BlockSpec((lblk, vocab, bsz), lambda l: (l + lb0, 0, 0))
    out_shape = jax.ShapeDtypeStruct((seq, vocab, bsz), jnp.float32)
    if out_prev is None:
        return pl.pallas_call(
            _lnmm_body,
            grid=grid,
            in_specs=common_in,
            out_specs=out_spec,
            out_shape=out_shape,
            compiler_params=pltpu.CompilerParams(
                vmem_limit_bytes=112 * 1024 * 1024
            ),
        )(h_t, lnw_p, lnb_p, fc_w_p, fc_bc)
    return pl.pallas_call(
        _lnmm_body2,
        grid=grid,
        in_specs=[pl.BlockSpec(memory_space=pl.ANY)] + common_in,
        out_specs=out_spec,
        out_shape=out_shape,
        input_output_aliases={0: 0},
    )(out_prev, h_t, lnw_p, lnb_p, fc_w_p, fc_bc)


def kernel(x, emb_table, ln_w, ln_b, fc_w, fc_b):
    bsz, seq = x.shape
    pad = HID_P - HIDDEN
    table_p = jnp.pad(emb_table, ((0, 0), (0, pad)))
    fc_w_p = jnp.pad(fc_w, ((0, 0), (0, pad)))
    lnw_p = jnp.pad(ln_w.reshape(1, -1), ((0, 0), (0, pad)))
    lnb_p = jnp.pad(ln_b.reshape(1, -1), ((0, 0), (0, pad)))
    fc_bc = fc_b.reshape(-1, 1)
    idx_t = x.T.reshape(-1)
    h_t = _sc_gather(idx_t, table_p, seq * bsz)
    out_t = _tc_lnmm(
        h_t, lnw_p, lnb_p, fc_w_p, fc_bc, seq, bsz, lblk=5, l0=0, out_prev=None
    )
    return out_t.transpose(2, 0, 1)
